# baseline (speedup 1.0000x reference)
import jax
import jax.numpy as jnp
from jax import lax
from jax.experimental import pallas as pl
from jax.experimental.pallas import tpu as pltpu
from jax.experimental.pallas import tpu_sc as plsc

N_FIELDS = 26
VOCAB1 = 100001
EMBED_DIM = 32
BATCH = 16384

NUM_CORES = 2
NUM_SUBCORES = 16
NW = NUM_CORES * NUM_SUBCORES
ROWS = N_FIELDS * EMBED_DIM
ROWS_PER_W = ROWS // NW
OUT_CHUNK = 4096
N_OCHUNK = BATCH // OUT_CHUNK
VPC = OUT_CHUNK // 16
UNROLL = 2


def _body(xt_hbm, tab_hbm, out_hbm, idx_v, row_v, out_v, sem0, sem1):
    wid = lax.axis_index("s") * NUM_CORES + lax.axis_index("c")
    lanes = lax.iota(jnp.int32, 16)
    sems = (sem0, sem1)
    descs = [None, None]
    f_prev = None

    for r in range(ROWS_PER_W):
        fd = wid * ROWS_PER_W + r
        f = fd // EMBED_DIM
        pltpu.sync_copy(tab_hbm.at[fd], row_v)
        if f_prev is None:
            pltpu.sync_copy(xt_hbm.at[f], idx_v)
        else:
            @pl.when(f != f_prev)
            def _load_idx(f=f):
                pltpu.sync_copy(xt_hbm.at[f], idx_v)
        f_prev = f

        v0 = row_v[pl.ds(0, 16)]
        row_v[pl.ds(0, 16)] = jnp.where(lanes == 0, 0.0, v0)

        for c in range(N_OCHUNK):
            b = (r * N_OCHUNK + c) % 2
            if descs[b] is not None:
                descs[b].wait()

            def gather_body(v, _, c=c, b=b):
                for u in range(UNROLL):
                    o = v * (16 * UNROLL) + u * 16
                    iv = idx_v[pl.ds(c * OUT_CHUNK + o, 16)]
                    out_v[b, pl.ds(o, 16)] = plsc.load_gather(row_v, [iv])
                return 0

            lax.fori_loop(0, VPC // UNROLL, gather_body, 0)
            descs[b] = pltpu.async_copy(
                out_v.at[b], out_hbm.at[fd, pl.ds(c * OUT_CHUNK, OUT_CHUNK)],
                sems[b])

    for d in descs:
        if d is not None:
            d.wait()


@jax.jit
def kernel(x, tables):
    xt = x.astype(jnp.int32).T
    tab = tables.transpose(0, 2, 1).reshape(ROWS, VOCAB1)
    emb = pl.kernel(
        _body,
        out_type=jax.ShapeDtypeStruct((ROWS, BATCH), jnp.float32),
        mesh=plsc.VectorSubcoreMesh(core_axis_name="c", subcore_axis_name="s",
                                    num_cores=NUM_CORES,
                                    num_subcores=NUM_SUBCORES),
        scratch_types=[
            pltpu.VMEM((BATCH,), jnp.int32),
            pltpu.VMEM((VOCAB1,), jnp.float32),
            pltpu.VMEM((2, OUT_CHUNK), jnp.float32),
            pltpu.SemaphoreType.DMA,
            pltpu.SemaphoreType.DMA,
        ],
        compiler_params=pltpu.CompilerParams(use_tc_tiling_on_sc=True,
                                             needs_layout_passes=False),
    )
    out_t = emb(xt, tab)
    return out_t.T


# speedup vs baseline: 1.6639x; 1.6639x over previous
import jax
import jax.numpy as jnp
from jax import lax
from jax.experimental import pallas as pl
from jax.experimental.pallas import tpu as pltpu
from jax.experimental.pallas import tpu_sc as plsc

N_FIELDS = 26
VOCAB1 = 100001
EMBED_DIM = 32
BATCH = 16384

NUM_CORES = 2
NUM_SUBCORES = 16
NW = NUM_CORES * NUM_SUBCORES
ROWS = N_FIELDS * EMBED_DIM
ROWS_PER_W = ROWS // NW
OUT_CHUNK = 4096
N_OCHUNK = BATCH // OUT_CHUNK
VPC = OUT_CHUNK // 16
UNROLL = 2
SKIP_GATHER = True


def _body(xt_hbm, tab_hbm, out_hbm, idx_v, row_v, out_v, sem0, sem1,
          gsem, *rsems):
    wid = lax.axis_index("s") * NUM_CORES + lax.axis_index("c")
    lanes = lax.iota(jnp.int32, 16)
    sems = (sem0, sem1)
    descs = [None, None]
    f_prev = None

    for r in range(ROWS_PER_W):
        fd = wid * ROWS_PER_W + r
        f = fd // EMBED_DIM
        pltpu.sync_copy(tab_hbm.at[fd], row_v)
        if f_prev is None:
            pltpu.sync_copy(xt_hbm.at[f], idx_v)
        else:
            @pl.when(f != f_prev)
            def _load_idx(f=f):
                pltpu.sync_copy(xt_hbm.at[f], idx_v)
        f_prev = f

        v0 = row_v[pl.ds(0, 16)]
        row_v[pl.ds(0, 16)] = jnp.where(lanes == 0, 0.0, v0)

        for c in range(N_OCHUNK):
            b = (r * N_OCHUNK + c) % 2
            if descs[b] is not None:
                descs[b].wait()

            @plsc.parallel_loop(0, VPC, 1, unroll=UNROLL)
            def _gather(v):
                iv = idx_v[pl.ds(c * OUT_CHUNK + v * 16, 16)]
                out_v[b, pl.ds(v * 16, 16)] = plsc.load_gather(row_v, [iv])
            descs[b] = pltpu.async_copy(
                out_v.at[b], out_hbm.at[fd, pl.ds(c * OUT_CHUNK, OUT_CHUNK)],
                sems[b])

    for d in descs:
        if d is not None:
            d.wait()


@jax.jit
def kernel(x, tables):
    xt = x.astype(jnp.int32).T
    tab = tables.transpose(0, 2, 1).reshape(ROWS, VOCAB1)
    emb = pl.kernel(
        _body,
        out_type=jax.ShapeDtypeStruct((ROWS, BATCH), jnp.float32),
        mesh=plsc.VectorSubcoreMesh(core_axis_name="c", subcore_axis_name="s",
                                    num_cores=NUM_CORES,
                                    num_subcores=NUM_SUBCORES),
        scratch_types=[
            pltpu.VMEM((BATCH,), jnp.int32),
            pltpu.VMEM((VOCAB1,), jnp.float32),
            pltpu.VMEM((2, OUT_CHUNK), jnp.float32),
            pltpu.SemaphoreType.DMA,
            pltpu.SemaphoreType.DMA,
            pltpu.SemaphoreType.DMA,
            pltpu.SemaphoreType.DMA,
            pltpu.SemaphoreType.DMA,
            pltpu.SemaphoreType.DMA,
        ],
        compiler_params=pltpu.CompilerParams(use_tc_tiling_on_sc=True,
                                             needs_layout_passes=False),
    )
    out_t = emb(xt, tab)
    return out_t.T


# speedup vs baseline: 1.9366x; 1.1639x over previous
import jax
import jax.numpy as jnp
from jax import lax
from jax.experimental import pallas as pl
from jax.experimental.pallas import tpu as pltpu
from jax.experimental.pallas import tpu_sc as plsc

N_FIELDS = 26
VOCAB1 = 100001
EMBED_DIM = 32
BATCH = 16384

NUM_CORES = 2
NUM_SUBCORES = 16
NW = NUM_CORES * NUM_SUBCORES
ROWS = N_FIELDS * EMBED_DIM
ROWS_PER_W = ROWS // NW
OUT_CHUNK = 4096
N_OCHUNK = BATCH // OUT_CHUNK
VPC = OUT_CHUNK // 16
UNROLL = 4
SKIP_GATHER = True


def _body(xt_hbm, tab_hbm, out_hbm, idx_v, row_v, out_v, sem0, sem1,
          gsem, *rsems):
    wid = lax.axis_index("s") * NUM_CORES + lax.axis_index("c")
    lanes = lax.iota(jnp.int32, 16)
    sems = (sem0, sem1)
    descs = [None, None]
    f_prev = None

    for r in range(ROWS_PER_W):
        fd = wid * ROWS_PER_W + r
        f = fd // EMBED_DIM
        pltpu.sync_copy(tab_hbm.at[fd], row_v)
        if f_prev is None:
            pltpu.sync_copy(xt_hbm.at[f], idx_v)
        else:
            @pl.when(f != f_prev)
            def _load_idx(f=f):
                pltpu.sync_copy(xt_hbm.at[f], idx_v)
        f_prev = f

        v0 = row_v[pl.ds(0, 16)]
        row_v[pl.ds(0, 16)] = jnp.where(lanes == 0, 0.0, v0)

        for c in range(N_OCHUNK):
            b = (r * N_OCHUNK + c) % 2
            if descs[b] is not None:
                descs[b].wait()

            @plsc.parallel_loop(0, VPC, 1, unroll=UNROLL)
            def _gather(v):
                iv = idx_v[pl.ds(c * OUT_CHUNK + v * 16, 16)]
                out_v[b, pl.ds(v * 16, 16)] = plsc.load_gather(row_v, [iv])
            descs[b] = pltpu.async_copy(
                out_v.at[b], out_hbm.at[fd, pl.ds(c * OUT_CHUNK, OUT_CHUNK)],
                sems[b])

    for d in descs:
        if d is not None:
            d.wait()


@jax.jit
def kernel(x, tables):
    xt = x.astype(jnp.int32).T
    tab = tables.transpose(0, 2, 1).reshape(ROWS, VOCAB1)
    emb = pl.kernel(
        _body,
        out_type=jax.ShapeDtypeStruct((ROWS, BATCH), jnp.float32),
        mesh=plsc.VectorSubcoreMesh(core_axis_name="c", subcore_axis_name="s",
                                    num_cores=NUM_CORES,
                                    num_subcores=NUM_SUBCORES),
        scratch_types=[
            pltpu.VMEM((BATCH,), jnp.int32),
            pltpu.VMEM((VOCAB1,), jnp.float32),
            pltpu.VMEM((2, OUT_CHUNK), jnp.float32),
            pltpu.SemaphoreType.DMA,
            pltpu.SemaphoreType.DMA,
            pltpu.SemaphoreType.DMA,
            pltpu.SemaphoreType.DMA,
            pltpu.SemaphoreType.DMA,
            pltpu.SemaphoreType.DMA,
        ],
        compiler_params=pltpu.CompilerParams(use_tc_tiling_on_sc=True,
                                             needs_layout_passes=False),
    )
    out_t = emb(xt, tab)
    return out_t.T


# speedup vs baseline: 1.9608x; 1.0125x over previous
import jax
import jax.numpy as jnp
from jax import lax
from jax.experimental import pallas as pl
from jax.experimental.pallas import tpu as pltpu
from jax.experimental.pallas import tpu_sc as plsc

N_FIELDS = 26
VOCAB1 = 100001
EMBED_DIM = 32
BATCH = 16384

NUM_CORES = 2
NUM_SUBCORES = 16
NW = NUM_CORES * NUM_SUBCORES
ROWS = N_FIELDS * EMBED_DIM
ROWS_PER_W = ROWS // NW
OUT_CHUNK = 4096
N_OCHUNK = BATCH // OUT_CHUNK
VPC = OUT_CHUNK // 16
UNROLL = 8
SKIP_GATHER = True


def _body(xt_hbm, tab_hbm, out_hbm, idx_v, row_v, out_v, sem0, sem1,
          gsem, *rsems):
    wid = lax.axis_index("s") * NUM_CORES + lax.axis_index("c")
    lanes = lax.iota(jnp.int32, 16)
    sems = (sem0, sem1)
    descs = [None, None]
    f_prev = None

    for r in range(ROWS_PER_W):
        fd = wid * ROWS_PER_W + r
        f = fd // EMBED_DIM
        pltpu.sync_copy(tab_hbm.at[fd], row_v)
        if f_prev is None:
            pltpu.sync_copy(xt_hbm.at[f], idx_v)
        else:
            @pl.when(f != f_prev)
            def _load_idx(f=f):
                pltpu.sync_copy(xt_hbm.at[f], idx_v)
        f_prev = f

        v0 = row_v[pl.ds(0, 16)]
        row_v[pl.ds(0, 16)] = jnp.where(lanes == 0, 0.0, v0)

        for c in range(N_OCHUNK):
            b = (r * N_OCHUNK + c) % 2
            if descs[b] is not None:
                descs[b].wait()

            @plsc.parallel_loop(0, VPC, 1, unroll=UNROLL)
            def _gather(v):
                iv = idx_v[pl.ds(c * OUT_CHUNK + v * 16, 16)]
                out_v[b, pl.ds(v * 16, 16)] = plsc.load_gather(row_v, [iv])
            descs[b] = pltpu.async_copy(
                out_v.at[b], out_hbm.at[fd, pl.ds(c * OUT_CHUNK, OUT_CHUNK)],
                sems[b])

    for d in descs:
        if d is not None:
            d.wait()


@jax.jit
def kernel(x, tables):
    xt = x.astype(jnp.int32).T
    tab = tables.transpose(0, 2, 1).reshape(ROWS, VOCAB1)
    emb = pl.kernel(
        _body,
        out_type=jax.ShapeDtypeStruct((ROWS, BATCH), jnp.float32),
        mesh=plsc.VectorSubcoreMesh(core_axis_name="c", subcore_axis_name="s",
                                    num_cores=NUM_CORES,
                                    num_subcores=NUM_SUBCORES),
        scratch_types=[
            pltpu.VMEM((BATCH,), jnp.int32),
            pltpu.VMEM((VOCAB1,), jnp.float32),
            pltpu.VMEM((2, OUT_CHUNK), jnp.float32),
            pltpu.SemaphoreType.DMA,
            pltpu.SemaphoreType.DMA,
            pltpu.SemaphoreType.DMA,
            pltpu.SemaphoreType.DMA,
            pltpu.SemaphoreType.DMA,
            pltpu.SemaphoreType.DMA,
        ],
        compiler_params=pltpu.CompilerParams(use_tc_tiling_on_sc=True,
                                             needs_layout_passes=False),
    )
    out_t = emb(xt, tab)
    return out_t.T


# speedup vs baseline: 1.9609x; 1.0001x over previous
import jax
import jax.numpy as jnp
from jax import lax
from jax.experimental import pallas as pl
from jax.experimental.pallas import tpu as pltpu
from jax.experimental.pallas import tpu_sc as plsc

N_FIELDS = 26
VOCAB1 = 100001
EMBED_DIM = 32
BATCH = 16384

NUM_CORES = 2
NUM_SUBCORES = 16
NW = NUM_CORES * NUM_SUBCORES
ROWS = N_FIELDS * EMBED_DIM
ROWS_PER_W = ROWS // NW
OUT_CHUNK = 4096
N_OCHUNK = BATCH // OUT_CHUNK
VPC = OUT_CHUNK // 16
UNROLL = 8


def _body(xt_hbm, tab_hbm, out_hbm, idx_v, row_v, out_v, sem0, sem1,
          gsem, *rsems):
    wid = lax.axis_index("s") * NUM_CORES + lax.axis_index("c")
    lanes = lax.iota(jnp.int32, 16)
    sems = (sem0, sem1)
    descs = [None, None]
    f_prev = None

    for r in range(ROWS_PER_W):
        fd = wid * ROWS_PER_W + r
        f = fd // EMBED_DIM
        pltpu.sync_copy(tab_hbm.at[fd], row_v)
        if f_prev is None:
            pltpu.sync_copy(xt_hbm.at[f], idx_v)
        else:
            @pl.when(f != f_prev)
            def _load_idx(f=f):
                pltpu.sync_copy(xt_hbm.at[f], idx_v)
        f_prev = f

        v0 = row_v[pl.ds(0, 16)]
        row_v[pl.ds(0, 16)] = jnp.where(lanes == 0, 0.0, v0)

        for c in range(N_OCHUNK):
            b = (r * N_OCHUNK + c) % 2
            if descs[b] is not None:
                descs[b].wait()

            @plsc.parallel_loop(0, VPC, 1, unroll=UNROLL)
            def _gather(v):
                iv = idx_v[pl.ds(c * OUT_CHUNK + v * 16, 16)]
                out_v[b, pl.ds(v * 16, 16)] = plsc.load_gather(row_v, [iv])
            descs[b] = pltpu.async_copy(
                out_v.at[b], out_hbm.at[fd, pl.ds(c * OUT_CHUNK, OUT_CHUNK)],
                sems[b])

    for d in descs:
        if d is not None:
            d.wait()


@jax.jit
def kernel(x, tables):
    xt = x.astype(jnp.int32).T
    tab = tables.transpose(0, 2, 1).reshape(ROWS, VOCAB1)
    emb = pl.kernel(
        _body,
        out_type=jax.ShapeDtypeStruct((ROWS, BATCH), jnp.float32),
        mesh=plsc.VectorSubcoreMesh(core_axis_name="c", subcore_axis_name="s",
                                    num_cores=NUM_CORES,
                                    num_subcores=NUM_SUBCORES),
        scratch_types=[
            pltpu.VMEM((BATCH,), jnp.int32),
            pltpu.VMEM((VOCAB1,), jnp.float32),
            pltpu.VMEM((2, OUT_CHUNK), jnp.float32),
            pltpu.SemaphoreType.DMA,
            pltpu.SemaphoreType.DMA,
            pltpu.SemaphoreType.DMA,
            pltpu.SemaphoreType.DMA,
            pltpu.SemaphoreType.DMA,
            pltpu.SemaphoreType.DMA,
        ],
        compiler_params=pltpu.CompilerParams(use_tc_tiling_on_sc=True,
                                             needs_layout_passes=False),
    )
    out_t = emb(xt, tab)
    return out_t.T


# speedup vs baseline: 1.9639x; 1.0015x over previous
import jax
import jax.numpy as jnp
from jax import lax
from jax.experimental import pallas as pl
from jax.experimental.pallas import tpu as pltpu
from jax.experimental.pallas import tpu_sc as plsc

N_FIELDS = 26
VOCAB1 = 100001
EMBED_DIM = 32
BATCH = 16384

NUM_CORES = 2
NUM_SUBCORES = 16
NW = NUM_CORES * NUM_SUBCORES
ROWS = N_FIELDS * EMBED_DIM
ROWS_PER_W = ROWS // NW
OUT_CHUNK = 4096
N_OCHUNK = BATCH // OUT_CHUNK
VPC = OUT_CHUNK // 16
UNROLL = 8


def _body(xt_hbm, tab_hbm, out_hbm, idx_v, row_v, out_v, sem0, sem1):
    wid = lax.axis_index("s") * NUM_CORES + lax.axis_index("c")
    lanes = lax.iota(jnp.int32, 16)
    sems = (sem0, sem1)
    descs = [None, None]
    f_prev = None

    for r in range(ROWS_PER_W):
        fd = wid * ROWS_PER_W + r
        f = fd // EMBED_DIM
        pltpu.sync_copy(tab_hbm.at[fd], row_v)
        if f_prev is None:
            pltpu.sync_copy(xt_hbm.at[f], idx_v)
        else:
            @pl.when(f != f_prev)
            def _load_idx(f=f):
                pltpu.sync_copy(xt_hbm.at[f], idx_v)
        f_prev = f

        v0 = row_v[pl.ds(0, 16)]
        row_v[pl.ds(0, 16)] = jnp.where(lanes == 0, 0.0, v0)

        for c in range(N_OCHUNK):
            b = (r * N_OCHUNK + c) % 2
            if descs[b] is not None:
                descs[b].wait()

            @plsc.parallel_loop(0, VPC, 1, unroll=UNROLL)
            def _gather(v):
                iv = idx_v[pl.ds(c * OUT_CHUNK + v * 16, 16)]
                out_v[b, pl.ds(v * 16, 16)] = plsc.load_gather(row_v, [iv])
            descs[b] = pltpu.async_copy(
                out_v.at[b], out_hbm.at[fd, pl.ds(c * OUT_CHUNK, OUT_CHUNK)],
                sems[b])

    for d in descs:
        if d is not None:
            d.wait()


@jax.jit
def kernel(x, tables):
    xt = x.astype(jnp.int32).T
    tab = tables.transpose(0, 2, 1).reshape(ROWS, VOCAB1)
    emb = pl.kernel(
        _body,
        out_type=jax.ShapeDtypeStruct((ROWS, BATCH), jnp.float32),
        mesh=plsc.VectorSubcoreMesh(core_axis_name="c", subcore_axis_name="s",
                                    num_cores=NUM_CORES,
                                    num_subcores=NUM_SUBCORES),
        scratch_types=[
            pltpu.VMEM((BATCH,), jnp.int32),
            pltpu.VMEM((VOCAB1,), jnp.float32),
            pltpu.VMEM((2, OUT_CHUNK), jnp.float32),
            pltpu.SemaphoreType.DMA,
            pltpu.SemaphoreType.DMA,
        ],
        compiler_params=pltpu.CompilerParams(use_tc_tiling_on_sc=True,
                                             needs_layout_passes=False),
    )
    out_t = emb(xt, tab)
    return out_t.T
